# baseline (speedup 1.0000x reference)
import functools

import jax
import jax.numpy as jnp
from jax.experimental import pallas as pl
from jax.experimental.pallas import tpu as pltpu

TOPK = 256


def _edge_body(q_ref, qh_ref, r_ref, t_ref, tm_ref, h_ref, wa_ref, ba_ref,
               watt_ref, batt_ref, wrule_ref, brule_ref, att_ref, msg_ref):
    r = r_ref[0]
    t = t_ref[0]
    tm = tm_ref[0]
    x = jnp.concatenate([r, t, tm], axis=-1)
    ne = jnp.dot(x, wa_ref[...]) + ba_ref[...]
    qb = q_ref[0]
    qa = jnp.concatenate([qb * ne, qb - ne, qb + ne], axis=-1)
    att1 = jax.nn.sigmoid(jnp.dot(qa, watt_ref[...]) + batt_ref[...])
    att2 = jax.nn.sigmoid(jnp.dot(h_ref[0], wrule_ref[...]) + brule_ref[...])
    att = (att1 + att2) / 2.0
    att_ref[0] = att
    msg_ref[0] = att * (qh_ref[0] + r + tm)


def _edge_pass(query_emd, q_head, r3, t3, tm3, hidden3, Wa, ba, Watt, batt, Wrule, brule):
    B, D = query_emd.shape
    N = r3.shape[1]
    full = lambda shape: pl.BlockSpec(shape, lambda b: tuple(0 for _ in shape))
    batch3 = pl.BlockSpec((1, N, D), lambda b: (b, 0, 0))
    att, msg = pl.pallas_call(
        _edge_body,
        grid=(B,),
        in_specs=[
            pl.BlockSpec((1, 1, D), lambda b: (b, 0, 0)),
            pl.BlockSpec((1, 1, D), lambda b: (b, 0, 0)),
            batch3, batch3, batch3, batch3,
            full((3 * D, D)),
            full((1, D)),
            full((3 * D, 1)),
            full((1, 1)),
            full((D, 1)),
            full((1, 1)),
        ],
        out_specs=[
            pl.BlockSpec((1, N, 1), lambda b: (b, 0, 0)),
            pl.BlockSpec((1, N, D), lambda b: (b, 0, 0)),
        ],
        out_shape=[
            jax.ShapeDtypeStruct((B, N, 1), jnp.float32),
            jax.ShapeDtypeStruct((B, N, D), jnp.float32),
        ],
    )(query_emd.reshape(B, 1, D), q_head.reshape(B, 1, D), r3, t3, tm3, hidden3,
      Wa.T, ba.reshape(1, D), Watt.T, batt.reshape(1, 1), Wrule.T, brule.reshape(1, 1))
    return att.reshape(B * N, 1), msg.reshape(B * N, D)


def kernel(q_head, q_rel, q_time, tail_nodes, tail_index, r_neighbor, t_neighbor, time_neighbor, hidden, tail_emd, batch_size, num_nodes, Wq, bq, Wa, ba, Watt, batt, Wrule, brule, Wout, bout):
    D = q_head.shape[-1]
    T = tail_nodes.shape[0]
    B = q_head.shape[0]
    N = r_neighbor.shape[1]
    size_zero = ((batch_size - B) + (num_nodes - N)).astype(jnp.float32) if hasattr(batch_size, "astype") else jnp.float32((batch_size - B) + (num_nodes - N))
    query_emd = jnp.concatenate([q_head, q_rel, q_time], axis=-1) @ Wq.T + bq
    att, message = _edge_pass(query_emd, q_head + size_zero, r_neighbor, t_neighbor,
                              time_neighbor, hidden.reshape(B, N, D),
                              Wa, ba, Watt, batt, Wrule, brule)
    tail_out = tail_emd + jax.ops.segment_sum(message, tail_index, num_segments=T)
    new_hidden = jax.ops.segment_sum(hidden, tail_index, num_segments=T)
    agg_att = jax.ops.segment_sum(att, tail_index, num_segments=T)[:, 0]
    nodes_batch = tail_nodes[:, 0]
    nodes_tail = tail_nodes[:, 1]
    nodes_time = tail_nodes[:, 2]

    def select_batch(i):
        mask = nodes_batch == i
        count = jnp.sum(mask)
        order = jax.lax.top_k(jnp.where(mask, -jnp.arange(T), -jnp.inf * jnp.ones(T)).astype(jnp.float32), TOPK)[1]
        valid = jnp.arange(TOPK) < count
        pad_tail = jnp.where(valid, nodes_tail[order], -jnp.ones((), nodes_tail.dtype))
        pad_time = jnp.where(valid, nodes_time[order], jnp.zeros((), nodes_time.dtype))
        pad_emd = jnp.where(valid[:, None], tail_out[order], jnp.zeros((), tail_out.dtype))
        pad_hid = jnp.where(valid[:, None], new_hidden[order], jnp.zeros((), new_hidden.dtype))
        idx = jax.lax.top_k(jnp.where(mask, agg_att, -jnp.inf), TOPK)[1]
        use_tk = count >= TOPK
        temp_tail = jnp.where(use_tk, nodes_tail[idx], pad_tail)
        temp_time = jnp.where(use_tk, nodes_time[idx], pad_time)
        temp_emd = jnp.where(use_tk, tail_out[idx], pad_emd)
        hid = jnp.where(use_tk, new_hidden[idx], pad_hid)
        return temp_tail, temp_time, temp_emd, hid

    tail_stack, time_stack, emd_stack, hidden_stack = jax.vmap(select_batch)(jnp.arange(B))
    new_nodes = jnp.stack([tail_stack, time_stack], axis=-1)
    tail_final = emd_stack @ Wout.T + bout
    return (new_nodes, tail_final, hidden_stack)


# speedup vs baseline: 4.6439x; 4.6439x over previous
import functools

import jax
import jax.numpy as jnp
from jax import lax
from jax.experimental import pallas as pl
from jax.experimental.pallas import tpu as pltpu
from jax.experimental.pallas import tpu_sc as plsc

TOPK = 256
_B, _N, _T, _D = 64, 2048, 65536, 128
_CAP = 4096
_SCAP = 512
_CH = 2048
_SENT = 0x7FFFFFFF


def _edge_body(q_ref, qh_ref, r_ref, t_ref, tm_ref, h_ref, wa_ref, ba_ref,
               watt_ref, batt_ref, wrule_ref, brule_ref, att_ref, msg_ref):
    r = r_ref[0]
    t = t_ref[0]
    tm = tm_ref[0]
    x = jnp.concatenate([r, t, tm], axis=-1)
    ne = jnp.dot(x, wa_ref[...]) + ba_ref[...]
    qb = q_ref[0]
    qa = jnp.concatenate([qb * ne, qb - ne, qb + ne], axis=-1)
    att1 = jax.nn.sigmoid(jnp.dot(qa, watt_ref[...]) + batt_ref[...])
    att2 = jax.nn.sigmoid(jnp.dot(h_ref[0], wrule_ref[...]) + brule_ref[...])
    att = (att1 + att2) / 2.0
    att_ref[0] = att
    msg_ref[0] = att * (qh_ref[0] + r + tm)


def _edge_pass(query_emd, q_head, r3, t3, tm3, hidden3, Wa, ba, Watt, batt, Wrule, brule):
    B, D = query_emd.shape
    N = r3.shape[1]
    full = lambda shape: pl.BlockSpec(shape, lambda b: tuple(0 for _ in shape))
    batch3 = pl.BlockSpec((1, N, D), lambda b: (b, 0, 0))
    att, msg = pl.pallas_call(
        _edge_body,
        grid=(B,),
        in_specs=[
            pl.BlockSpec((1, 1, D), lambda b: (b, 0, 0)),
            pl.BlockSpec((1, 1, D), lambda b: (b, 0, 0)),
            batch3, batch3, batch3, batch3,
            full((3 * D, D)),
            full((1, D)),
            full((3 * D, 1)),
            full((1, 1)),
            full((D, 1)),
            full((1, 1)),
        ],
        out_specs=[
            pl.BlockSpec((1, N, 1), lambda b: (b, 0, 0)),
            pl.BlockSpec((1, N, D), lambda b: (b, 0, 0)),
        ],
        out_shape=[
            jax.ShapeDtypeStruct((B, N, 1), jnp.float32),
            jax.ShapeDtypeStruct((B, N, D), jnp.float32),
        ],
    )(query_emd.reshape(B, 1, D), q_head.reshape(B, 1, D), r3, t3, tm3, hidden3,
      Wa.T, ba.reshape(1, D), Watt.T, batt.reshape(1, 1), Wrule.T, brule.reshape(1, 1))
    return att.reshape(B * N, 1), msg.reshape(B * N, D)


def _vsum(m):
    r = plsc.all_reduce_population_count(m)
    return r[0] if getattr(r, "ndim", 0) else r


def _topk_body(agg_hbm, nb_hbm, ntail_hbm, ntime_hbm,
               fid_out, tail_out, time_out, valid_out,
               k0, i0, k1, i1, nbv, aggv, sk, si,
               ordv, ord2v, fidv, validv, gtail, gtime, sem):
    nc = 2
    wid = lax.axis_index("s") * nc + lax.axis_index("c")
    kmax = jnp.uint32(0xFFFFFFFF)
    sent = jnp.int32(_SENT)

    def pre_body(v, _):
        for kr, ir in ((k0, i0), (k1, i1)):
            kr[pl.ds(v * 16, 16)] = jnp.full((16,), kmax, jnp.uint32)
            ir[pl.ds(v * 16, 16)] = jnp.full((16,), sent, jnp.int32)
        return 0
    lax.fori_loop(0, _CAP // 16 + 1, pre_body, 0)

    b0 = wid * 2
    b1 = wid * 2 + 1

    def chunk_body(ci, carry):
        cc = lax.rem(ci + wid, _T // _CH)
        base = cc * _CH
        pltpu.sync_copy(nb_hbm.at[pl.ds(base, _CH)], nbv)
        pltpu.sync_copy(agg_hbm.at[pl.ds(base, _CH)], aggv)

        def vec_body(v, cur):
            cur0, cur1 = cur
            nb16 = nbv[pl.ds(v * 16, 16)]
            a16 = aggv[pl.ds(v * 16, 16)]
            kc = ~lax.bitcast_convert_type(a16, jnp.uint32)
            tid = base + v * 16 + lax.iota(jnp.int32, 16)
            m0 = nb16 == b0
            n0 = _vsum(m0)

            @pl.when((n0 > 0) & (cur0 <= _CAP - 16))
            def _():
                plsc.store_compressed(k0.at[pl.ds(cur0, 16)], kc, mask=m0)
                plsc.store_compressed(i0.at[pl.ds(cur0, 16)], tid, mask=m0)
            cur0 = jnp.where(cur0 <= _CAP - 16, cur0 + n0, cur0)

            m1 = nb16 == b1
            n1 = _vsum(m1)

            @pl.when((n1 > 0) & (cur1 <= _CAP - 16))
            def _():
                plsc.store_compressed(k1.at[pl.ds(cur1, 16)], kc, mask=m1)
                plsc.store_compressed(i1.at[pl.ds(cur1, 16)], tid, mask=m1)
            cur1 = jnp.where(cur1 <= _CAP - 16, cur1 + n1, cur1)
            return (cur0, cur1)

        return lax.fori_loop(0, _CH // 16, vec_body, carry)

    ncand0, ncand1 = lax.fori_loop(0, _T // _CH, chunk_body,
                                   (jnp.int32(0), jnp.int32(0)))

    for slot in range(2):
        kr, ir = ((k0, i0), (k1, i1))[slot]
        n_ = (ncand0, ncand1)[slot]
        b = (b0, b1)[slot]
        nv = (n_ + 15) // 16

        def bit_body(bi, p):
            t0 = p + (jnp.uint32(1) << (31 - bi).astype(jnp.uint32))

            def cnt_body(v, acc):
                kv = kr[pl.ds(v * 16, 16)]
                return acc + _vsum(kv < t0)
            cnt = lax.fori_loop(0, nv, cnt_body, jnp.int32(0))
            return jnp.where(cnt >= TOPK, p, t0)
        theta = lax.fori_loop(0, 32, bit_body, jnp.uint32(0))

        def spre_body(v, _):
            sk[pl.ds(v * 16, 16)] = jnp.full((16,), kmax, jnp.uint32)
            si[pl.ds(v * 16, 16)] = jnp.full((16,), sent, jnp.int32)
            return 0
        lax.fori_loop(0, _SCAP // 16 + 1, spre_body, 0)

        def sv_body(v, scur):
            kv = kr[pl.ds(v * 16, 16)]
            iv = ir[pl.ds(v * 16, 16)]
            m = (kv <= theta) & (iv != sent)
            nm = _vsum(m)

            @pl.when((nm > 0) & (scur <= _SCAP - 16))
            def _():
                plsc.store_compressed(sk.at[pl.ds(scur, 16)], kv, mask=m)
                plsc.store_compressed(si.at[pl.ds(scur, 16)], iv, mask=m)
            return jnp.where(scur <= _SCAP - 16, scur + nm, scur)
        scur = lax.fori_loop(0, nv, sv_body, jnp.int32(0))

        def opre_body(v, _):
            ordv[pl.ds(v * 16, 16)] = jnp.zeros((16,), jnp.int32)
            ord2v[pl.ds(v * 16, 16)] = jnp.zeros((16,), jnp.int32)
            return 0
        lax.fori_loop(0, TOPK // 16, opre_body, 0)

        nsv = (jnp.minimum(scur, _SCAP) + 15) // 16

        def rk_body(j, _):
            kj = sk[pl.ds(j, 16)][0]
            ij = si[pl.ds(j, 16)][0]

            def rb(v, acc):
                kv = sk[pl.ds(v * 16, 16)]
                iv = si[pl.ds(v * 16, 16)]
                lt = (kv < kj) | ((kv == kj) & (iv < ij))
                return acc + _vsum(lt)
            rank = lax.fori_loop(0, nsv, rb, jnp.int32(0))

            @pl.when(rank < TOPK)
            def _():
                lane0 = lax.iota(jnp.int32, 16) == 0
                plsc.store_scatter(ordv, [jnp.full((16,), rank, jnp.int32)],
                                   jnp.full((16,), ij, jnp.int32), mask=lane0)
            return 0
        lax.fori_loop(0, jnp.minimum(scur, _SCAP), rk_body, 0)

        @pl.when(n_ < TOPK)
        def _():
            def fb_body(j, _):
                ij = ir[pl.ds(j, 16)][0]

                def rb2(v, acc):
                    iv = ir[pl.ds(v * 16, 16)]
                    return acc + _vsum(iv < ij)
                rank2 = lax.fori_loop(0, nv, rb2, jnp.int32(0))
                lane0 = lax.iota(jnp.int32, 16) == 0
                plsc.store_scatter(ord2v, [jnp.full((16,), rank2, jnp.int32)],
                                   jnp.full((16,), ij, jnp.int32), mask=lane0)
                return 0
            lax.fori_loop(0, n_, fb_body, 0)

        use_tk = n_ >= TOPK

        def mg_body(v, _):
            kg = v * 16 + lax.iota(jnp.int32, 16)
            o1 = ordv[pl.ds(v * 16, 16)]
            o2 = ord2v[pl.ds(v * 16, 16)]
            fid = jnp.where(use_tk, o1, o2)
            vld = use_tk | (kg < n_)
            fidv[pl.ds(v * 16, 16)] = jnp.where(vld, fid, 0)
            validv[pl.ds(v * 16, 16)] = jnp.where(vld, 1, 0).astype(jnp.int32)
            return 0
        lax.fori_loop(0, TOPK // 16, mg_body, 0)

        pltpu.async_copy(ntail_hbm.at[fidv], gtail, sem).wait()
        pltpu.async_copy(ntime_hbm.at[fidv], gtime, sem).wait()

        def sel_body(v, _):
            vld = validv[pl.ds(v * 16, 16)] != 0
            gtail[pl.ds(v * 16, 16)] = jnp.where(vld, gtail[pl.ds(v * 16, 16)],
                                                 jnp.int32(-1))
            gtime[pl.ds(v * 16, 16)] = jnp.where(vld, gtime[pl.ds(v * 16, 16)],
                                                 jnp.int32(0))
            return 0
        lax.fori_loop(0, TOPK // 16, sel_body, 0)

        pltpu.sync_copy(fidv, fid_out.at[b])
        pltpu.sync_copy(validv, valid_out.at[b])
        pltpu.sync_copy(gtail, tail_out.at[b])
        pltpu.sync_copy(gtime, time_out.at[b])


def _topk_pass(agg, nb, ntail, ntime):
    mesh = plsc.VectorSubcoreMesh(core_axis_name="c", subcore_axis_name="s")
    i32 = jnp.int32
    f = pl.kernel(
        _topk_body,
        mesh=mesh,
        compiler_params=pltpu.CompilerParams(needs_layout_passes=False),
        out_type=[
            jax.ShapeDtypeStruct((_B, TOPK), i32),
            jax.ShapeDtypeStruct((_B, TOPK), i32),
            jax.ShapeDtypeStruct((_B, TOPK), i32),
            jax.ShapeDtypeStruct((_B, TOPK), i32),
        ],
        scratch_types=[
            pltpu.VMEM((_CAP + 16,), jnp.uint32),
            pltpu.VMEM((_CAP + 16,), i32),
            pltpu.VMEM((_CAP + 16,), jnp.uint32),
            pltpu.VMEM((_CAP + 16,), i32),
            pltpu.VMEM((_CH,), i32),
            pltpu.VMEM((_CH,), jnp.float32),
            pltpu.VMEM((_SCAP + 16,), jnp.uint32),
            pltpu.VMEM((_SCAP + 16,), i32),
            pltpu.VMEM((TOPK,), i32),
            pltpu.VMEM((TOPK,), i32),
            pltpu.VMEM((TOPK,), i32),
            pltpu.VMEM((TOPK,), i32),
            pltpu.VMEM((TOPK,), i32),
            pltpu.VMEM((TOPK,), i32),
            pltpu.SemaphoreType.DMA,
        ],
    )
    return f(agg, nb, ntail, ntime)


_PB = 2048
_G = 64
_EPT = 8192
_STR = 2048
_NPASS = (_B * TOPK) // (2 * _PB)


def _scat_body(tidx_hbm, msg_hbm, hid_hbm, fid_hbm, val_hbm, temd_hbm,
               emd_out, hid_out,
               inv, fidbuf, mi, mp, mp2, rbufm, rbufh, zbuf,
               macc, hacc, sem, sem2):
    c = lax.axis_index("c")
    s = lax.axis_index("s")
    ebase = s * _EPT
    dump_l = jnp.int32(_PB) + lax.rem(s, 8)

    def zb(v, _):
        r = v // (_D // 16)
        k = lax.rem(v, _D // 16)
        zbuf[r, pl.ds(k * 16, 16)] = jnp.zeros((16,), jnp.float32)
        return 0
    lax.fori_loop(0, 16 * (_D // 16), zb, 0)

    def iv(v, _):
        inv[pl.ds(v * 16, 16)] = jnp.full((16,), -1, jnp.int32)
        return 0
    lax.fori_loop(0, _T // 16, iv, 0)

    def ichunk(cc, _):
        g0 = cc * _STR
        pltpu.sync_copy(fid_hbm.at[pl.ds(g0, _STR)], fidbuf)
        pltpu.sync_copy(val_hbm.at[pl.ds(g0, _STR)], mi.at[pl.ds(0, _STR)])

        def ivec(v, _):
            fv = fidbuf[pl.ds(v * 16, 16)]
            vv = mi[pl.ds(v * 16, 16)]
            pos = g0 + v * 16 + lax.iota(jnp.int32, 16)
            m = vv != 0
            plsc.store_scatter(inv, [fv], pos, mask=m)
            return 0
        lax.fori_loop(0, _STR // 16, ivec, 0)
        return 0
    lax.fori_loop(0, (_B * TOPK) // _STR, ichunk, 0)

    for p in range(_NPASS):
        pbase = (p * 2) * _PB + c * _PB

        def zr(v, _):
            base = s * (_PB // 16) + v * 16
            pltpu.sync_copy(zbuf, macc.at[pl.ds(base, 16)])
            pltpu.sync_copy(zbuf, hacc.at[pl.ds(base, 16)])
            return 0
        lax.fori_loop(0, _PB // 16 // 16, zr, 0)

        @pl.when(s == 0)
        def _():
            pltpu.sync_copy(zbuf, macc.at[pl.ds(_PB, 16)])
            pltpu.sync_copy(zbuf, hacc.at[pl.ds(_PB, 16)])
        plsc.subcore_barrier()

        def strip(st, _):
            e0 = ebase + st * _STR
            pltpu.sync_copy(tidx_hbm.at[pl.ds(e0, _STR)], fidbuf)

            def evec(v, nm):
                tv = fidbuf[pl.ds(v * 16, 16)]
                pg = plsc.load_gather(inv, [tv])
                m = (pg >= pbase) & (pg < pbase + _PB)
                p16 = pg - pbase
                n = _vsum(m)

                @pl.when(n > 0)
                def _():
                    eid = e0 + v * 16 + lax.iota(jnp.int32, 16)
                    plsc.store_compressed(mi.at[pl.ds(nm, 16)], eid, mask=m)
                    plsc.store_compressed(mp.at[pl.ds(nm, 16)], p16, mask=m)
                return nm + n
            nm = lax.fori_loop(0, _STR // 16, evec, jnp.int32(0))

            def padk(k, _):
                mi[pl.ds(nm + k * 16, 16)] = jnp.zeros((16,), jnp.int32)
                mp[pl.ds(nm + k * 16, 16)] = jnp.full((16,), dump_l, jnp.int32)
                return 0
            lax.fori_loop(0, _G // 16, padk, 0)

            nch = (nm + _G - 1) // _G

            def gchunk(j, _):
                def cprow(k, _):
                    mp2[j, pl.ds(k * 16, 16)] = mp[pl.ds(j * _G + k * 16, 16)]
                    return 0
                lax.fori_loop(0, _G // 16, cprow, 0)
                cm = pltpu.async_copy(msg_hbm.at[mi.at[pl.ds(j * _G, _G)]], rbufm, sem)
                ch = pltpu.async_copy(hid_hbm.at[mi.at[pl.ds(j * _G, _G)]], rbufh, sem2)
                cm.wait()
                ch.wait()
                pltpu.sync_copy(rbufm, macc.at[mp2.at[j]], add=True)
                pltpu.sync_copy(rbufh, hacc.at[mp2.at[j]], add=True)
                return 0
            lax.fori_loop(0, nch, gchunk, 0)
            return 0
        lax.fori_loop(0, _EPT // _STR, strip, 0)
        plsc.subcore_barrier()

        for half in range(_PB // 16 // _G):
            lrow = s * (_PB // 16) + half * _G
            g0 = pbase + lrow
            pltpu.sync_copy(val_hbm.at[pl.ds(g0, _G)], mi.at[pl.ds(0, _G)])
            pltpu.sync_copy(hacc.at[pl.ds(lrow, _G)], rbufh)

            def hrow(r, _):
                vf = jnp.where(mi[pl.ds(r, 16)][0] > 0, 1.0, 0.0)
                for k in range(_D // 16):
                    rbufh[r, pl.ds(k * 16, 16)] = rbufh[r, pl.ds(k * 16, 16)] * vf
                return 0
            lax.fori_loop(0, _G, hrow, 0)
            pltpu.sync_copy(rbufh, hid_out.at[pl.ds(g0, _G)])

            pltpu.sync_copy(fid_hbm.at[pl.ds(g0, _G)], fidbuf.at[pl.ds(0, _G)])
            pltpu.async_copy(temd_hbm.at[fidbuf.at[pl.ds(0, _G)]], rbufh, sem).wait()
            pltpu.sync_copy(macc.at[pl.ds(lrow, _G)], rbufm)

            def mrow(r, _):
                vf = jnp.where(mi[pl.ds(r, 16)][0] > 0, 1.0, 0.0)
                for k in range(_D // 16):
                    rbufm[r, pl.ds(k * 16, 16)] = (rbufm[r, pl.ds(k * 16, 16)]
                                                   + rbufh[r, pl.ds(k * 16, 16)]) * vf
                return 0
            lax.fori_loop(0, _G, mrow, 0)
            pltpu.sync_copy(rbufm, emd_out.at[pl.ds(g0, _G)])
        plsc.subcore_barrier()


def _scatter_pass(tidx, msg, hid, fid_flat, val_flat, temd):
    mesh = plsc.VectorSubcoreMesh(core_axis_name="c", subcore_axis_name="s")
    i32 = jnp.int32
    f32 = jnp.float32
    f = pl.kernel(
        _scat_body,
        mesh=mesh,
        compiler_params=pltpu.CompilerParams(needs_layout_passes=False),
        out_type=[
            jax.ShapeDtypeStruct((_B * TOPK, _D), f32),
            jax.ShapeDtypeStruct((_B * TOPK, _D), f32),
        ],
        scratch_types=[
            pltpu.VMEM((_T,), i32),
            pltpu.VMEM((_STR,), i32),
            pltpu.VMEM((_STR + _G + 16,), i32),
            pltpu.VMEM((_STR + _G + 16,), i32),
            pltpu.VMEM((_STR // _G + 1, _G), i32),
            pltpu.VMEM((_G, _D), f32),
            pltpu.VMEM((_G, _D), f32),
            pltpu.VMEM((16, _D), f32),
            pltpu.VMEM_SHARED((_PB + 16, _D), f32),
            pltpu.VMEM_SHARED((_PB + 16, _D), f32),
            pltpu.SemaphoreType.DMA,
            pltpu.SemaphoreType.DMA,
        ],
    )
    return f(tidx, msg, hid, fid_flat, val_flat, temd)


def _fin_body(x_ref, w_ref, b_ref, o_ref):
    o_ref[...] = jnp.dot(x_ref[...], w_ref[...]) + b_ref[...]


def _final_matmul(x, Wout, bout):
    M = x.shape[0]
    BLK = 2048
    return pl.pallas_call(
        _fin_body,
        grid=(M // BLK,),
        in_specs=[
            pl.BlockSpec((BLK, _D), lambda i: (i, 0)),
            pl.BlockSpec((_D, _D), lambda i: (0, 0)),
            pl.BlockSpec((1, _D), lambda i: (0, 0)),
        ],
        out_specs=pl.BlockSpec((BLK, _D), lambda i: (i, 0)),
        out_shape=jax.ShapeDtypeStruct((M, _D), jnp.float32),
    )(x, Wout.T, bout.reshape(1, _D))


def kernel(q_head, q_rel, q_time, tail_nodes, tail_index, r_neighbor, t_neighbor, time_neighbor, hidden, tail_emd, batch_size, num_nodes, Wq, bq, Wa, ba, Watt, batt, Wrule, brule, Wout, bout):
    D = q_head.shape[-1]
    T = tail_nodes.shape[0]
    B = q_head.shape[0]
    N = r_neighbor.shape[1]
    size_zero = ((batch_size - B) + (num_nodes - N)).astype(jnp.float32)
    query_emd = jnp.concatenate([q_head, q_rel, q_time], axis=-1) @ Wq.T + bq
    att, message = _edge_pass(query_emd, q_head + size_zero, r_neighbor, t_neighbor,
                              time_neighbor, hidden.reshape(B, N, D),
                              Wa, ba, Watt, batt, Wrule, brule)
    agg_att = jax.ops.segment_sum(att, tail_index, num_segments=T)[:, 0]

    nb = tail_nodes[:, 0]
    ntl = tail_nodes[:, 1]
    ntm = tail_nodes[:, 2]
    fid, otail, otime, valid = _topk_pass(agg_att, nb, ntl, ntm)
    new_nodes = jnp.stack([otail, otime], axis=-1)

    emd_sel, hid_sel = _scatter_pass(tail_index, message, hidden,
                                     fid.reshape(B * TOPK), valid.reshape(B * TOPK),
                                     tail_emd)
    tail_final = _final_matmul(emd_sel, Wout, bout).reshape(B, TOPK, D)
    hidden_stack = hid_sel.reshape(B, TOPK, D)
    return (new_nodes, tail_final, hidden_stack)
